# Initial kernel scaffold; baseline (speedup 1.0000x reference)
import jax
import jax.numpy as jnp
from jax.experimental import pallas as pl


def kernel(obs, table, W, b):
    raise NotImplementedError("write your pallas kernel here")


# speedup vs baseline: 4.0413x; 4.0413x over previous
import functools

import jax
import jax.numpy as jnp
from jax import lax
from jax.experimental import pallas as pl
from jax.experimental.pallas import tpu as pltpu
from jax.experimental.pallas import tpu_sc as plsc

EMBED = 16
NUM_CORES = 2
NUM_SUBCORES = 16
NUM_WORKERS = NUM_CORES * NUM_SUBCORES
CHUNK = 128
NBUF = 5
MM_BLOCK = 512


def _sc_gather(table, idx_flat):
    num_indices = idx_flat.shape[0]
    per_worker = num_indices // NUM_WORKERS
    num_chunks = per_worker // CHUNK
    assert per_worker % CHUNK == 0 and num_chunks % NBUF == 0

    mesh = plsc.VectorSubcoreMesh(core_axis_name="c", subcore_axis_name="s")

    scratch = [pltpu.VMEM((per_worker,), jnp.int32)]
    scratch += [pltpu.VMEM((CHUNK, EMBED), jnp.float32) for _ in range(NBUF)]
    scratch += [pltpu.SemaphoreType.DMA for _ in range(NBUF)]

    @functools.partial(
        pl.kernel,
        mesh=mesh,
        out_type=jax.ShapeDtypeStruct((num_indices, EMBED), table.dtype),
        scratch_types=scratch,
        compiler_params=pltpu.CompilerParams(use_tc_tiling_on_sc=False),
    )
    def gather_kernel(table_hbm, idx_hbm, out_hbm, idx_v, *rows_and_sems):
        rows = rows_and_sems[:NBUF]
        sems = rows_and_sems[NBUF:]
        wid = lax.axis_index("s") * NUM_CORES + lax.axis_index("c")
        base = wid * per_worker
        pltpu.sync_copy(idx_hbm.at[pl.ds(base, per_worker)], idx_v)

        def start(chunk, b):
            pltpu.async_copy(
                table_hbm.at[idx_v.at[pl.ds(chunk * CHUNK, CHUNK)]],
                rows[b],
                sems[b],
            )

        for b in range(NBUF):
            start(b, b)

        @pl.loop(0, num_chunks, step=NBUF)
        def _(c0):
            for b in range(NBUF):
                c = c0 + b
                pltpu.make_async_copy(
                    table_hbm.at[idx_v.at[pl.ds(0, CHUNK)]], rows[b], sems[b]
                ).wait()
                pltpu.sync_copy(
                    rows[b], out_hbm.at[pl.ds(base + c * CHUNK, CHUNK)]
                )
                nxt = c + NBUF

                @pl.when(nxt < num_chunks)
                def _():
                    start(nxt, b)

    return gather_kernel(table, idx_flat)


def _tc_matmul(x, W, b):
    m, k = x.shape
    n = W.shape[1]

    def mm_kernel(x_ref, w_ref, b_ref, o_ref):
        o_ref[...] = (
            jnp.dot(x_ref[...], w_ref[...], preferred_element_type=jnp.float32)
            + b_ref[...]
        )

    return pl.pallas_call(
        mm_kernel,
        grid=(m // MM_BLOCK,),
        in_specs=[
            pl.BlockSpec((MM_BLOCK, k), lambda i: (i, 0)),
            pl.BlockSpec((k, n), lambda i: (0, 0)),
            pl.BlockSpec((1, n), lambda i: (0, 0)),
        ],
        out_specs=pl.BlockSpec((MM_BLOCK, n), lambda i: (i, 0)),
        out_shape=jax.ShapeDtypeStruct((m, n), jnp.float32),
    )(x, W, b.reshape(1, n))


def kernel(obs, table, W, b):
    batch, seq, obs_dim = obs.shape
    num_indices = batch * seq * obs_dim
    idx_flat = obs.reshape(num_indices)
    emb = _sc_gather(table, idx_flat)
    emb_flat = emb.reshape(batch * seq, obs_dim * EMBED)
    out = _tc_matmul(emb_flat, W, b)
    return out.reshape(batch, seq, -1)


# speedup vs baseline: 4.5676x; 1.1302x over previous
import functools

import jax
import jax.numpy as jnp
from jax import lax
from jax.experimental import pallas as pl
from jax.experimental.pallas import tpu as pltpu
from jax.experimental.pallas import tpu_sc as plsc

EMBED = 16
NUM_CORES = 2
NUM_SUBCORES = 16
NUM_WORKERS = NUM_CORES * NUM_SUBCORES
CHUNK = 128
NBUF = 5
GROUP = 4
MM_BLOCK = 512


def _sc_gather(table, idx_flat):
    num_indices = idx_flat.shape[0]
    per_worker = num_indices // NUM_WORKERS
    num_chunks = per_worker // CHUNK
    assert per_worker % CHUNK == 0 and num_chunks % NBUF == 0

    mesh = plsc.VectorSubcoreMesh(core_axis_name="c", subcore_axis_name="s")

    scratch = [pltpu.VMEM((per_worker,), jnp.int32)]
    scratch += [pltpu.VMEM((CHUNK, EMBED), jnp.float32) for _ in range(NBUF)]
    scratch += [pltpu.SemaphoreType.DMA for _ in range(NBUF)]

    @functools.partial(
        pl.kernel,
        mesh=mesh,
        out_type=jax.ShapeDtypeStruct((num_indices, EMBED), table.dtype),
        scratch_types=scratch,
        compiler_params=pltpu.CompilerParams(use_tc_tiling_on_sc=False),
    )
    def gather_kernel(table_hbm, idx_hbm, out_hbm, idx_v, *rows_and_sems):
        rows = rows_and_sems[:NBUF]
        sems = rows_and_sems[NBUF:]
        wid = lax.axis_index("s") * NUM_CORES + lax.axis_index("c")
        base = wid * per_worker
        pltpu.sync_copy(idx_hbm.at[pl.ds(base, per_worker)], idx_v)

        def start(chunk, b):
            pltpu.async_copy(
                table_hbm.at[idx_v.at[pl.ds(chunk * CHUNK, CHUNK)]],
                rows[b],
                sems[b],
            )

        for b in range(NBUF):
            start(b, b)

        @pl.loop(0, num_chunks, step=NBUF)
        def _(c0):
            for b in range(NBUF):
                c = c0 + b
                pltpu.make_async_copy(
                    table_hbm.at[idx_v.at[pl.ds(0, CHUNK)]], rows[b], sems[b]
                ).wait()
                pltpu.sync_copy(
                    rows[b], out_hbm.at[pl.ds(base + c * CHUNK, CHUNK)]
                )
                nxt = c + NBUF

                @pl.when(nxt < num_chunks)
                def _():
                    start(nxt, b)

    return gather_kernel(table, idx_flat)


def _tc_matmul(x2, W, b):
    out_dim = W.shape[1]
    num_rows = x2.shape[0] * 128 // (W.shape[0])
    xrows_per_block = MM_BLOCK * W.shape[0] // 128
    groups = MM_BLOCK // GROUP

    t2 = jnp.zeros((GROUP * W.shape[0] // 128 * 128, GROUP * out_dim), x2.dtype)
    for a in range(GROUP):
        t2 = lax.dynamic_update_slice(t2, W, (a * W.shape[0], a * out_dim))
    b_tiled = jnp.tile(b, GROUP).reshape(1, GROUP * out_dim)

    def mm_kernel(x_ref, t2_ref, b_ref, o_ref):
        xg = x_ref[...].reshape(groups, xrows_per_block * 128 // groups)
        out = (
            jnp.dot(xg, t2_ref[...], preferred_element_type=jnp.float32)
            + b_ref[...]
        )
        o_ref[...] = out.reshape(MM_BLOCK, out_dim)

    return pl.pallas_call(
        mm_kernel,
        grid=(num_rows // MM_BLOCK,),
        in_specs=[
            pl.BlockSpec((xrows_per_block, 128), lambda i: (i, 0)),
            pl.BlockSpec(t2.shape, lambda i: (0, 0)),
            pl.BlockSpec(b_tiled.shape, lambda i: (0, 0)),
        ],
        out_specs=pl.BlockSpec((MM_BLOCK, out_dim), lambda i: (i, 0)),
        out_shape=jax.ShapeDtypeStruct((num_rows, out_dim), jnp.float32),
    )(x2, t2, b_tiled)


def kernel(obs, table, W, b):
    batch, seq, obs_dim = obs.shape
    num_indices = batch * seq * obs_dim
    idx_flat = obs.reshape(num_indices)
    emb = _sc_gather(table, idx_flat)
    emb128 = emb.reshape(num_indices * EMBED // 128, 128)
    out = _tc_matmul(emb128, W, b)
    return out.reshape(batch, seq, -1)
